# baseline (speedup 1.0000x reference)
import functools

import jax
import jax.numpy as jnp
from jax import lax
from jax.experimental import pallas as pl
from jax.experimental.pallas import tpu as pltpu

B, S, DIM = 1, 2048, 2048
H, G, DK, DV = 16, 4, 128, 128
HPG = H // G
L, D, LSEL, NSEL, W = 32, 16, 64, 16, 512
NCMP = (S - L) // D + 1
NSB = S // LSEL
HMULT = 4

QB = 256
KB = 256
NQ = S // QB
NKT = S // KB
NEG = -1e9
SCALE = 1.0 / (DK ** 0.5)
NPROJ = H * DK + G * DK + G * DV + 128
GOFF = H * DK + G * DK + G * DV
HI = jax.lax.Precision.HIGHEST

f32 = jnp.float32
bf16 = jnp.bfloat16


def _iota(shape, dim):
    return lax.broadcasted_iota(jnp.int32, shape, dim)


def _proj_body(x_ref, w_ref, nw_ref, o_ref):
    xb = x_ref[...]
    ms = jnp.mean(xb * xb, axis=-1, keepdims=True)
    xn = xb * lax.rsqrt(ms + 1e-6) * nw_ref[...]
    o_ref[...] = jnp.dot(xn.astype(bf16), w_ref[...],
                         preferred_element_type=f32)


def _attn_body(qkvg_ref, o_ref, kvbf_ref, kcmp_ref, vcmp_ref, selm_ref):
    qi = pl.program_id(0)
    t0 = qi * QB
    t = t0 + _iota((QB, 1), 0)

    @pl.when(qi == 0)
    def _init():
        kvbf_ref[...] = qkvg_ref[:, H * DK:H * DK + 2 * G * DK].astype(bf16)
        ci = _iota((128, S), 0)
        si = _iota((128, S), 1)
        P = jnp.where((si >= ci * D) & (si < ci * D + L), f32(1.0 / L),
                      f32(0.0))
        for g in range(G):
            kf = qkvg_ref[:, H * DK + g * DK:H * DK + (g + 1) * DK]
            vf = qkvg_ref[:, H * DK + G * DK + g * DV:
                          H * DK + G * DK + (g + 1) * DV]
            kcmp_ref[g * 128:(g + 1) * 128, :] = jnp.dot(
                P, kf, preferred_element_type=f32, precision=HI)
            vcmp_ref[g * 128:(g + 1) * 128, :] = jnp.dot(
                P, vf, preferred_element_type=f32, precision=HI)

    ci = _iota((QB, 128), 1)
    vis = ((ci * D + L - 1) <= t) & (ci < NCMP)
    visf = vis.astype(f32)

    Mseg = jnp.where((_iota((128, 128), 0) // 4 == _iota((128, 128), 1))
                     & (_iota((128, 128), 0) < NCMP), f32(1.0), f32(0.0))

    jlane = _iota((QB, 128), 1)

    for g in range(G):
        kcmp = kcmp_ref[g * 128:(g + 1) * 128, :]
        vcmp = vcmp_ref[g * 128:(g + 1) * 128, :]

        p_imp = jnp.zeros((QB, 128), f32)
        out_cmp = []
        for h in range(HPG):
            hg = g * HPG + h
            qh = qkvg_ref[pl.ds(t0, QB), hg * DK:(hg + 1) * DK]
            sc = lax.dot_general(qh, kcmp, (((1,), (1,)), ((), ())),
                                 precision=HI,
                                 preferred_element_type=f32) * SCALE
            sc = jnp.where(vis, sc, NEG)
            m = jnp.max(sc, axis=-1, keepdims=True)
            p = jnp.exp(sc - m)
            p = p / jnp.sum(p, axis=-1, keepdims=True)
            p = p * visf
            p_imp = p_imp + p
            out_cmp.append(jnp.dot(p.astype(bf16),
                                   vcmp.astype(bf16),
                                   preferred_element_type=f32))

        selr = jnp.dot(p_imp, Mseg, precision=HI, preferred_element_type=f32)
        allowed = (jlane * LSEL <= t) & (jlane < NSB)
        force = (jlane == 0) | (jlane == t // LSEL)
        s = jnp.where(allowed, selr + 1e9 * force.astype(f32), NEG)
        cnt = jnp.zeros((QB, 128), jnp.int32)
        for jj in range(NSB):
            cnt = cnt + (s[:, jj:jj + 1] > s).astype(jnp.int32)
        mask_blk = ((cnt < NSEL) & (jlane < NSB)).astype(f32)

        def _mk(kb, _):
            rep = jnp.where(
                ((kb * KB + _iota((128, KB), 1)) // LSEL) == _iota((128, KB), 0),
                f32(1.0), f32(0.0))
            selm_ref[kb] = jnp.dot(mask_blk.astype(bf16), rep.astype(bf16),
                                   preferred_element_type=f32)
            return 0

        lax.fori_loop(0, qi + 1, _mk, 0)

        for h in range(HPG):
            hg = g * HPG + h
            qh_bf = qkvg_ref[pl.ds(t0, QB), hg * DK:(hg + 1) * DK].astype(bf16)

            def _tile(kb, carry):
                ms_, ls_, as_, mw_, lw_, aw_ = carry
                kblk = kvbf_ref[pl.ds(kb * KB, KB), g * DK:(g + 1) * DK]
                vblk = kvbf_ref[pl.ds(kb * KB, KB),
                                G * DK + g * DV:G * DK + (g + 1) * DV]
                qk = lax.dot_general(qh_bf, kblk, (((1,), (1,)), ((), ())),
                                     preferred_element_type=f32) * SCALE
                pcol = kb * KB + _iota((QB, KB), 1)
                qk = jnp.where(pcol <= t, qk, NEG)

                ssel = jnp.where(selm_ref[kb] > 0.5, qk, NEG)
                m2 = jnp.maximum(ms_, jnp.max(ssel, axis=-1, keepdims=True))
                live = (m2 > -5e8).astype(f32)
                e = jnp.exp(ssel - m2) * live
                fo = jnp.exp(ms_ - m2)
                ls_ = ls_ * fo + jnp.sum(e, axis=-1, keepdims=True)
                as_ = as_ * fo + jnp.dot(e.astype(bf16), vblk,
                                         preferred_element_type=f32)
                ms_ = m2

                def _win(c):
                    mw, lw, aw = c
                    sw = jnp.where(pcol > t - W, qk, NEG)
                    m2w = jnp.maximum(mw, jnp.max(sw, axis=-1, keepdims=True))
                    livew = (m2w > -5e8).astype(f32)
                    ew = jnp.exp(sw - m2w) * livew
                    fw = jnp.exp(mw - m2w)
                    lw = lw * fw + jnp.sum(ew, axis=-1, keepdims=True)
                    aw = aw * fw + jnp.dot(ew.astype(bf16), vblk,
                                           preferred_element_type=f32)
                    return m2w, lw, aw

                mw_, lw_, aw_ = lax.cond(kb >= qi - 2, _win, lambda c: c,
                                         (mw_, lw_, aw_))
                return ms_, ls_, as_, mw_, lw_, aw_

            z1 = jnp.full((QB, 1), NEG)
            z0 = jnp.zeros((QB, 1), f32)
            za = jnp.zeros((QB, DV), f32)
            ms_, ls_, as_, mw_, lw_, aw_ = lax.fori_loop(
                0, qi + 1, _tile, (z1, z0, za, z1, z0, za))
            out_sel = as_ / ls_
            out_win = aw_ / lw_

            def _gate(e):
                gcol = qkvg_ref[pl.ds(t0, QB), GOFF + hg * 3 + e:
                                GOFF + hg * 3 + e + 1]
                return 1.0 / (1.0 + jnp.exp(-gcol))

            out_h = (_gate(0) * out_cmp[h] + _gate(1) * out_sel
                     + _gate(2) * out_win)
            o_ref[:, hg * DV:(hg + 1) * DV] = out_h


def _c1_body(a_ref, wo_ref, x_ref, nw_ref, x2_ref, h_ref):
    x2 = jnp.dot(a_ref[...].astype(bf16), wo_ref[...],
                 preferred_element_type=f32) + x_ref[...]
    x2_ref[...] = x2
    ms = jnp.mean(x2 * x2, axis=-1, keepdims=True)
    h_ref[...] = (x2 * lax.rsqrt(ms + 1e-6) * nw_ref[...]).astype(bf16)


def _c2_body(h_ref, x2_ref, w1_ref, w2_ref, o_ref):
    j = pl.program_id(0)
    for mi in range(S // QB):
        sl = slice(mi * QB, (mi + 1) * QB)
        a = jnp.dot(h_ref[sl, :], w1_ref[...], preferred_element_type=f32)
        a = a * (1.0 / (1.0 + jnp.exp(-a)))
        contrib = jnp.dot(a.astype(bf16), w2_ref[...],
                          preferred_element_type=f32)

        @pl.when(j == 0)
        def _():
            o_ref[sl, :] = x2_ref[sl, :] + contrib

        @pl.when(j > 0)
        def _():
            o_ref[sl, :] = o_ref[sl, :] + contrib


def kernel(x, norm1_w, Wq, Wk, Wv, Wg, Wo, norm2_w, W1, W2):
    xs = x.reshape(S, DIM)
    Wall = jnp.concatenate(
        [Wq, Wk, Wv, jnp.pad(Wg, ((0, 0), (0, 128 - H * 3)))],
        axis=1).astype(bf16)

    qkvg = pl.pallas_call(
        _proj_body,
        grid=(NQ,),
        in_specs=[
            pl.BlockSpec((QB, DIM), lambda i: (i, 0)),
            pl.BlockSpec((DIM, NPROJ), lambda i: (0, 0)),
            pl.BlockSpec((1, DIM), lambda i: (0, 0)),
        ],
        out_specs=pl.BlockSpec((QB, NPROJ), lambda i: (i, 0)),
        out_shape=jax.ShapeDtypeStruct((S, NPROJ), f32),
    )(xs, Wall, norm1_w.reshape(1, DIM))

    attn = pl.pallas_call(
        _attn_body,
        grid=(NQ,),
        in_specs=[pl.BlockSpec((S, NPROJ), lambda i: (0, 0))],
        out_specs=pl.BlockSpec((QB, H * DV), lambda i: (i, 0)),
        out_shape=jax.ShapeDtypeStruct((S, H * DV), f32),
        scratch_shapes=[
            pltpu.VMEM((S, 2 * G * DK), bf16),
            pltpu.VMEM((G * 128, DK), f32),
            pltpu.VMEM((G * 128, DV), f32),
            pltpu.VMEM((NKT, QB, KB), f32),
        ],
    )(qkvg)

    x2, hbf = pl.pallas_call(
        _c1_body,
        grid=(NQ,),
        in_specs=[
            pl.BlockSpec((QB, H * DV), lambda i: (i, 0)),
            pl.BlockSpec((H * DV, DIM), lambda i: (0, 0)),
            pl.BlockSpec((QB, DIM), lambda i: (i, 0)),
            pl.BlockSpec((1, DIM), lambda i: (0, 0)),
        ],
        out_specs=[pl.BlockSpec((QB, DIM), lambda i: (i, 0)),
                   pl.BlockSpec((QB, DIM), lambda i: (i, 0))],
        out_shape=[jax.ShapeDtypeStruct((S, DIM), f32),
                   jax.ShapeDtypeStruct((S, DIM), bf16)],
    )(attn, Wo.astype(bf16), xs, norm2_w.reshape(1, DIM))

    NC2 = 16
    CH = HMULT * DIM // NC2
    y = pl.pallas_call(
        _c2_body,
        grid=(NC2,),
        in_specs=[
            pl.BlockSpec((S, DIM), lambda j: (0, 0)),
            pl.BlockSpec((S, DIM), lambda j: (0, 0)),
            pl.BlockSpec((DIM, CH), lambda j: (0, j)),
            pl.BlockSpec((CH, DIM), lambda j: (j, 0)),
        ],
        out_specs=pl.BlockSpec((S, DIM), lambda j: (0, 0)),
        out_shape=jax.ShapeDtypeStruct((S, DIM), f32),
    )(hbf, x2, W1.astype(bf16), W2.astype(bf16))

    return y.reshape(B, S, DIM)


# speedup vs baseline: 1.4467x; 1.4467x over previous
import functools

import jax
import jax.numpy as jnp
from jax import lax
from jax.experimental import pallas as pl
from jax.experimental.pallas import tpu as pltpu

B, S, DIM = 1, 2048, 2048
H, G, DK, DV = 16, 4, 128, 128
HPG = H // G
L, D, LSEL, NSEL, W = 32, 16, 64, 16, 512
NCMP = (S - L) // D + 1
NSB = S // LSEL
HMULT = 4

QB = 256
KB = 256
NQ = S // QB
NKT = S // KB
NEG = -1e9
SCALE = 1.0 / (DK ** 0.5)
NPROJ = H * DK + G * DK + G * DV + 128
GOFF = H * DK + G * DK + G * DV
HI = jax.lax.Precision.HIGHEST

f32 = jnp.float32
bf16 = jnp.bfloat16


def _iota(shape, dim):
    return lax.broadcasted_iota(jnp.int32, shape, dim)


def _proj_body(x_ref, w_ref, nw_ref, o_ref):
    xb = x_ref[...]
    ms = jnp.mean(xb * xb, axis=-1, keepdims=True)
    xn = xb * lax.rsqrt(ms + 1e-6) * nw_ref[...]
    o_ref[...] = jnp.dot(xn.astype(bf16), w_ref[...],
                         preferred_element_type=f32)


def _attn_body(qkvg_ref, o_ref, kvbf_ref, kcmp_ref, vcmp_ref, selm_ref):
    qi = pl.program_id(0)
    t0 = qi * QB
    t = t0 + _iota((QB, 1), 0)

    @pl.when(qi == 0)
    def _init():
        kvbf_ref[...] = qkvg_ref[:, H * DK:H * DK + 2 * G * DK].astype(bf16)
        ci = _iota((128, S), 0)
        si = _iota((128, S), 1)
        P = jnp.where((si >= ci * D) & (si < ci * D + L), f32(1.0 / L),
                      f32(0.0))
        for g in range(G):
            kf = qkvg_ref[:, H * DK + g * DK:H * DK + (g + 1) * DK]
            vf = qkvg_ref[:, H * DK + G * DK + g * DV:
                          H * DK + G * DK + (g + 1) * DV]
            kcmp_ref[g * 128:(g + 1) * 128, :] = jnp.dot(
                P, kf, preferred_element_type=f32, precision=HI)
            vcmp_ref[g * 128:(g + 1) * 128, :] = jnp.dot(
                P, vf, preferred_element_type=f32, precision=HI)

    QH = HPG * QB
    t4 = t0 + _iota((QH, 1), 0) % QB
    ci4 = _iota((QH, 128), 1)
    vis4 = ((ci4 * D + L - 1) <= t4) & (ci4 < NCMP)
    visf4 = vis4.astype(f32)

    Mseg = jnp.where((_iota((128, 128), 0) // 4 == _iota((128, 128), 1))
                     & (_iota((128, 128), 0) < NCMP), f32(1.0), f32(0.0))

    jlane = _iota((QB, 128), 1)

    for g in range(G):
        kcmp = kcmp_ref[g * 128:(g + 1) * 128, :]
        vcmp = vcmp_ref[g * 128:(g + 1) * 128, :]

        q4 = jnp.concatenate(
            [qkvg_ref[pl.ds(t0, QB), (g * HPG + h) * DK:(g * HPG + h + 1) * DK]
             for h in range(HPG)], axis=0)
        q4bf = (q4 * SCALE).astype(bf16)

        sc = lax.dot_general(q4, kcmp, (((1,), (1,)), ((), ())),
                             precision=HI,
                             preferred_element_type=f32) * SCALE
        sc = jnp.where(vis4, sc, NEG)
        m = jnp.max(sc, axis=-1, keepdims=True)
        p = jnp.exp(sc - m)
        p = p / jnp.sum(p, axis=-1, keepdims=True)
        p = p * visf4
        out_cmp4 = jnp.dot(p.astype(bf16), vcmp.astype(bf16),
                           preferred_element_type=f32)
        p_imp = (p[0:QB] + p[QB:2 * QB] + p[2 * QB:3 * QB] + p[3 * QB:4 * QB])

        selr = jnp.dot(p_imp, Mseg, precision=HI, preferred_element_type=f32)
        allowed = (jlane * LSEL <= t) & (jlane < NSB)
        force = (jlane == 0) | (jlane == t // LSEL)
        s = jnp.where(allowed, selr + 1e9 * force.astype(f32), NEG)
        cnt = jnp.zeros((QB, 128), jnp.int32)
        for jj in range(NSB):
            cnt = cnt + (s[:, jj:jj + 1] > s).astype(jnp.int32)
        mask_blk = ((cnt < NSEL) & (jlane < NSB)).astype(f32)

        def _mk(kb, _):
            rep = jnp.where(
                ((kb * KB + _iota((128, KB), 1)) // LSEL) == _iota((128, KB), 0),
                f32(1.0), f32(0.0))
            mt = jnp.dot(mask_blk.astype(bf16), rep.astype(bf16),
                         preferred_element_type=f32)
            for h in range(HPG):
                selm_ref[kb, h * QB:(h + 1) * QB, :] = mt
            return 0

        lax.fori_loop(0, qi + 1, _mk, 0)

        def _kv(kb):
            kblk = kvbf_ref[pl.ds(kb * KB, KB), g * DK:(g + 1) * DK]
            vblk = kvbf_ref[pl.ds(kb * KB, KB),
                            G * DK + g * DV:G * DK + (g + 1) * DV]
            return kblk, vblk

        def _upd(sx, mx, lx, ax, vblk, guard):
            m2 = jnp.maximum(mx, jnp.max(sx, axis=-1, keepdims=True))
            e = jnp.exp(sx - m2)
            if guard:
                e = e * (m2 > -5e8).astype(f32)
            fo = jnp.exp(mx - m2)
            lx = lx * fo + jnp.sum(e, axis=-1, keepdims=True)
            ax = ax * fo + jnp.dot(e.astype(bf16), vblk,
                                   preferred_element_type=f32)
            return m2, lx, ax

        def _tile(kb, carry):
            ms_, ls_, as_ = carry
            kblk, vblk = _kv(kb)
            qk = lax.dot_general(q4bf, kblk, (((1,), (1,)), ((), ())),
                                 preferred_element_type=f32)
            ssel = jnp.where(selm_ref[kb] > 0.5, qk, NEG)
            return _upd(ssel, ms_, ls_, as_, vblk, False)

        z1 = jnp.full((QH, 1), NEG)
        z0 = jnp.zeros((QH, 1), f32)
        za = jnp.zeros((QH, DV), f32)
        ms_, ls_, as_ = lax.fori_loop(0, jnp.maximum(qi - 2, 0), _tile,
                                      (z1, z0, za))
        mw_, lw_, aw_ = z1, z0, za

        for r in range(3):
            kbr = qi - 2 + r
            kbc = jnp.maximum(kbr, 0)
            kblk, vblk = _kv(kbc)
            qk = lax.dot_general(q4bf, kblk, (((1,), (1,)), ((), ())),
                                 preferred_element_type=f32)
            if r < 2:
                qk = jnp.where(kbr >= 0, qk, NEG)
            else:
                pcol = kbc * KB + _iota((QH, KB), 1)
                qk = jnp.where(pcol <= t4, qk, NEG)
            ssel = jnp.where(selm_ref[kbc] > 0.5, qk, NEG)
            ms_, ls_, as_ = _upd(ssel, ms_, ls_, as_, vblk, r < 2)
            if r == 0:
                pcol = kbc * KB + _iota((QH, KB), 1)
                sw = jnp.where(pcol > t4 - W, qk, NEG)
            else:
                sw = qk
            mw_, lw_, aw_ = _upd(sw, mw_, lw_, aw_, vblk, r < 2)

        out_sel4 = as_ / ls_
        out_win4 = aw_ / lw_

        for h in range(HPG):
            hg = g * HPG + h
            row = slice(h * QB, (h + 1) * QB)

            def _gate(e):
                gcol = qkvg_ref[pl.ds(t0, QB), GOFF + hg * 3 + e:
                                GOFF + hg * 3 + e + 1]
                return 1.0 / (1.0 + jnp.exp(-gcol))

            out_h = (_gate(0) * out_cmp4[row] + _gate(1) * out_sel4[row]
                     + _gate(2) * out_win4[row])
            o_ref[:, hg * DV:(hg + 1) * DV] = out_h


def _c1_body(a_ref, wo_ref, x_ref, nw_ref, x2_ref, h_ref):
    x2 = jnp.dot(a_ref[...].astype(bf16), wo_ref[...],
                 preferred_element_type=f32) + x_ref[...]
    x2_ref[...] = x2
    ms = jnp.mean(x2 * x2, axis=-1, keepdims=True)
    h_ref[...] = (x2 * lax.rsqrt(ms + 1e-6) * nw_ref[...]).astype(bf16)


def _c2_body(h_ref, x2_ref, w1_ref, w2_ref, o_ref):
    j = pl.program_id(0)
    for mi in range(S // QB):
        sl = slice(mi * QB, (mi + 1) * QB)
        a = jnp.dot(h_ref[sl, :], w1_ref[...], preferred_element_type=f32)
        a = a * (1.0 / (1.0 + jnp.exp(-a)))
        contrib = jnp.dot(a.astype(bf16), w2_ref[...],
                          preferred_element_type=f32)

        @pl.when(j == 0)
        def _():
            o_ref[sl, :] = x2_ref[sl, :] + contrib

        @pl.when(j > 0)
        def _():
            o_ref[sl, :] = o_ref[sl, :] + contrib


def kernel(x, norm1_w, Wq, Wk, Wv, Wg, Wo, norm2_w, W1, W2):
    xs = x.reshape(S, DIM)
    Wall = jnp.concatenate(
        [Wq, Wk, Wv, jnp.pad(Wg, ((0, 0), (0, 128 - H * 3)))],
        axis=1).astype(bf16)

    qkvg = pl.pallas_call(
        _proj_body,
        grid=(NQ,),
        in_specs=[
            pl.BlockSpec((QB, DIM), lambda i: (i, 0)),
            pl.BlockSpec((DIM, NPROJ), lambda i: (0, 0)),
            pl.BlockSpec((1, DIM), lambda i: (0, 0)),
        ],
        out_specs=pl.BlockSpec((QB, NPROJ), lambda i: (i, 0)),
        out_shape=jax.ShapeDtypeStruct((S, NPROJ), f32),
    )(xs, Wall, norm1_w.reshape(1, DIM))

    attn = pl.pallas_call(
        _attn_body,
        grid=(NQ,),
        in_specs=[pl.BlockSpec((S, NPROJ), lambda i: (0, 0))],
        out_specs=pl.BlockSpec((QB, H * DV), lambda i: (i, 0)),
        out_shape=jax.ShapeDtypeStruct((S, H * DV), f32),
        scratch_shapes=[
            pltpu.VMEM((S, 2 * G * DK), bf16),
            pltpu.VMEM((G * 128, DK), f32),
            pltpu.VMEM((G * 128, DV), f32),
            pltpu.VMEM((NKT, HPG * QB, KB), f32),
        ],
    )(qkvg)

    x2, hbf = pl.pallas_call(
        _c1_body,
        grid=(NQ,),
        in_specs=[
            pl.BlockSpec((QB, H * DV), lambda i: (i, 0)),
            pl.BlockSpec((H * DV, DIM), lambda i: (0, 0)),
            pl.BlockSpec((QB, DIM), lambda i: (i, 0)),
            pl.BlockSpec((1, DIM), lambda i: (0, 0)),
        ],
        out_specs=[pl.BlockSpec((QB, DIM), lambda i: (i, 0)),
                   pl.BlockSpec((QB, DIM), lambda i: (i, 0))],
        out_shape=[jax.ShapeDtypeStruct((S, DIM), f32),
                   jax.ShapeDtypeStruct((S, DIM), bf16)],
    )(attn, Wo.astype(bf16), xs, norm2_w.reshape(1, DIM))

    NC2 = 16
    CH = HMULT * DIM // NC2
    y = pl.pallas_call(
        _c2_body,
        grid=(NC2,),
        in_specs=[
            pl.BlockSpec((S, DIM), lambda j: (0, 0)),
            pl.BlockSpec((S, DIM), lambda j: (0, 0)),
            pl.BlockSpec((DIM, CH), lambda j: (0, j)),
            pl.BlockSpec((CH, DIM), lambda j: (j, 0)),
        ],
        out_specs=pl.BlockSpec((S, DIM), lambda j: (0, 0)),
        out_shape=jax.ShapeDtypeStruct((S, DIM), f32),
    )(hbf, x2, W1.astype(bf16), W2.astype(bf16))

    return y.reshape(B, S, DIM)


# speedup vs baseline: 1.4718x; 1.0173x over previous
import functools

import jax
import jax.numpy as jnp
from jax import lax
from jax.experimental import pallas as pl
from jax.experimental.pallas import tpu as pltpu

B, S, DIM = 1, 2048, 2048
H, G, DK, DV = 16, 4, 128, 128
HPG = H // G
L, D, LSEL, NSEL, W = 32, 16, 64, 16, 512
NCMP = (S - L) // D + 1
NSB = S // LSEL
HMULT = 4

QB = 256
KB = 256
NQ = S // QB
NKT = S // KB
NEG = -1e9
SCALE = 1.0 / (DK ** 0.5)
NPROJ = H * DK + G * DK + G * DV + 128
GOFF = H * DK + G * DK + G * DV
HI = jax.lax.Precision.HIGHEST

f32 = jnp.float32
bf16 = jnp.bfloat16


def _iota(shape, dim):
    return lax.broadcasted_iota(jnp.int32, shape, dim)


def _proj_body(x_ref, w_ref, nw_ref, o_ref):
    xb = x_ref[...]
    ms = jnp.mean(xb * xb, axis=-1, keepdims=True)
    xn = xb * lax.rsqrt(ms + 1e-6) * nw_ref[...]
    o_ref[...] = jnp.dot(xn.astype(bf16), w_ref[...],
                         preferred_element_type=f32)


def _attn_body(qg_ref, kv_ref, o_ref, kvbf_ref, kcmp_ref, vcmp_ref, selm_ref):
    qi = pl.program_id(0)
    t0 = qi * QB
    t = t0 + _iota((QB, 1), 0)

    @pl.when(qi == 0)
    def _init():
        kvbf_ref[...] = kv_ref[...].astype(bf16)
        ci = _iota((128, S), 0)
        si = _iota((128, S), 1)
        P = jnp.where((si >= ci * D) & (si < ci * D + L), f32(1.0 / L),
                      f32(0.0))
        for g in range(G):
            kf = kv_ref[:, g * DK:(g + 1) * DK]
            vf = kv_ref[:, G * DK + g * DV:G * DK + (g + 1) * DV]
            kcmp_ref[g * 128:(g + 1) * 128, :] = jnp.dot(
                P, kf, preferred_element_type=f32, precision=HI)
            vcmp_ref[g * 128:(g + 1) * 128, :] = jnp.dot(
                P, vf, preferred_element_type=f32, precision=HI)

    QH = HPG * QB
    t4 = t0 + _iota((QH, 1), 0) % QB
    ci4 = _iota((QH, 128), 1)
    vis4 = ((ci4 * D + L - 1) <= t4) & (ci4 < NCMP)
    visf4 = vis4.astype(f32)

    Mseg = jnp.where((_iota((128, 128), 0) // 4 == _iota((128, 128), 1))
                     & (_iota((128, 128), 0) < NCMP), f32(1.0), f32(0.0))

    jlane = _iota((QB, 128), 1)

    for g in range(G):
        kcmp = kcmp_ref[g * 128:(g + 1) * 128, :]
        vcmp = vcmp_ref[g * 128:(g + 1) * 128, :]

        q4 = jnp.concatenate(
            [qg_ref[:, (g * HPG + h) * DK:(g * HPG + h + 1) * DK]
             for h in range(HPG)], axis=0)
        q4bf = (q4 * SCALE).astype(bf16)

        sc = lax.dot_general(q4, kcmp, (((1,), (1,)), ((), ())),
                             precision=HI,
                             preferred_element_type=f32) * SCALE
        sc = jnp.where(vis4, sc, NEG)
        m = jnp.max(sc, axis=-1, keepdims=True)
        p = jnp.exp(sc - m)
        p = p / jnp.sum(p, axis=-1, keepdims=True)
        p = p * visf4
        out_cmp4 = jnp.dot(p.astype(bf16), vcmp.astype(bf16),
                           preferred_element_type=f32)
        p_imp = (p[0:QB] + p[QB:2 * QB] + p[2 * QB:3 * QB] + p[3 * QB:4 * QB])

        selr = jnp.dot(p_imp, Mseg, precision=HI, preferred_element_type=f32)
        allowed = (jlane * LSEL <= t) & (jlane < NSB)
        force = (jlane == 0) | (jlane == t // LSEL)
        s = jnp.where(allowed, selr + 1e9 * force.astype(f32), NEG)
        cnt = jnp.zeros((QB, 128), jnp.int32)
        for jj in range(NSB):
            cnt = cnt + (s[:, jj:jj + 1] > s).astype(jnp.int32)
        mask_blk = ((cnt < NSEL) & (jlane < NSB)).astype(f32)

        def _mk(kb, _):
            rep = jnp.where(
                ((kb * KB + _iota((128, KB), 1)) // LSEL) == _iota((128, KB), 0),
                f32(1.0), f32(0.0))
            mt = jnp.dot(mask_blk.astype(bf16), rep.astype(bf16),
                         preferred_element_type=f32)
            bias = ((mt - 1.0) * 1e9).astype(bf16)
            for h in range(HPG):
                selm_ref[kb, h * QB:(h + 1) * QB, :] = bias
            return 0

        lax.fori_loop(0, qi + 1, _mk, 0)

        def _kv(kb):
            kblk = kvbf_ref[pl.ds(kb * KB, KB), g * DK:(g + 1) * DK]
            vblk = kvbf_ref[pl.ds(kb * KB, KB),
                            G * DK + g * DV:G * DK + (g + 1) * DV]
            return kblk, vblk

        def _upd(sx, mx, lx, ax, vblk, guard):
            m2 = jnp.maximum(mx, jnp.max(sx, axis=-1, keepdims=True))
            e = jnp.exp(sx - m2)
            if guard:
                e = e * (m2 > -5e8).astype(f32)
            fo = jnp.exp(mx - m2)
            lx = lx * fo + jnp.sum(e, axis=-1, keepdims=True)
            ax = ax * fo + jnp.dot(e.astype(bf16), vblk,
                                   preferred_element_type=f32)
            return m2, lx, ax

        def _tile(kb, carry):
            ms_, ls_, as_ = carry
            kblk, vblk = _kv(kb)
            qk = lax.dot_general(q4bf, kblk, (((1,), (1,)), ((), ())),
                                 preferred_element_type=f32)
            ssel = qk + selm_ref[kb]
            return _upd(ssel, ms_, ls_, as_, vblk, False)

        z1 = jnp.full((QH, 1), NEG)
        z0 = jnp.zeros((QH, 1), f32)
        za = jnp.zeros((QH, DV), f32)
        ms_, ls_, as_ = lax.fori_loop(0, jnp.maximum(qi - 2, 0), _tile,
                                      (z1, z0, za))
        mw_, lw_, aw_ = z1, z0, za

        for r in range(3):
            kbr = qi - 2 + r
            kbc = jnp.maximum(kbr, 0)
            kblk, vblk = _kv(kbc)
            qk = lax.dot_general(q4bf, kblk, (((1,), (1,)), ((), ())),
                                 preferred_element_type=f32)
            if r < 2:
                qk = jnp.where(kbr >= 0, qk, NEG)
            else:
                pcol = kbc * KB + _iota((QH, KB), 1)
                qk = jnp.where(pcol <= t4, qk, NEG)
            ssel = qk + selm_ref[kbc]
            ms_, ls_, as_ = _upd(ssel, ms_, ls_, as_, vblk, r < 2)
            if r == 0:
                pcol = kbc * KB + _iota((QH, KB), 1)
                sw = jnp.where(pcol > t4 - W, qk, NEG)
            else:
                sw = qk
            mw_, lw_, aw_ = _upd(sw, mw_, lw_, aw_, vblk, r < 2)

        out_sel4 = as_ / ls_
        out_win4 = aw_ / lw_

        for h in range(HPG):
            hg = g * HPG + h
            row = slice(h * QB, (h + 1) * QB)

            def _gate(e):
                gcol = qg_ref[:, GOFF + hg * 3 + e:GOFF + hg * 3 + e + 1]
                return 1.0 / (1.0 + jnp.exp(-gcol))

            out_h = (_gate(0) * out_cmp4[row] + _gate(1) * out_sel4[row]
                     + _gate(2) * out_win4[row])
            o_ref[:, hg * DV:(hg + 1) * DV] = out_h


def _c1_body(a_ref, wo_ref, x_ref, nw_ref, x2_ref, h_ref):
    x2 = jnp.dot(a_ref[...].astype(bf16), wo_ref[...],
                 preferred_element_type=f32) + x_ref[...]
    x2_ref[...] = x2
    ms = jnp.mean(x2 * x2, axis=-1, keepdims=True)
    h_ref[...] = (x2 * lax.rsqrt(ms + 1e-6) * nw_ref[...]).astype(bf16)


def _c2_body(h_ref, x2_ref, w1_ref, w2_ref, o_ref):
    j = pl.program_id(0)
    for mi in range(8):
        sl = slice(mi * 256, (mi + 1) * 256)
        a = jnp.dot(h_ref[sl, :], w1_ref[...], preferred_element_type=f32)
        a = a * (1.0 / (1.0 + jnp.exp(-a)))
        contrib = jnp.dot(a.astype(bf16), w2_ref[...],
                          preferred_element_type=f32)

        @pl.when(j == 0)
        def _():
            o_ref[sl, :] = x2_ref[sl, :] + contrib

        @pl.when(j > 0)
        def _():
            o_ref[sl, :] = o_ref[sl, :] + contrib


def kernel(x, norm1_w, Wq, Wk, Wv, Wg, Wo, norm2_w, W1, W2):
    xs = x.reshape(S, DIM)
    Wall = jnp.concatenate(
        [Wq, Wk, Wv, jnp.pad(Wg, ((0, 0), (0, 128 - H * 3)))],
        axis=1).astype(bf16)

    qkvg = pl.pallas_call(
        _proj_body,
        grid=(NQ,),
        in_specs=[
            pl.BlockSpec((QB, DIM), lambda i: (i, 0)),
            pl.BlockSpec((DIM, NPROJ), lambda i: (0, 0)),
            pl.BlockSpec((1, DIM), lambda i: (0, 0)),
        ],
        out_specs=pl.BlockSpec((QB, NPROJ), lambda i: (i, 0)),
        out_shape=jax.ShapeDtypeStruct((S, NPROJ), f32),
    )(xs, Wall, norm1_w.reshape(1, DIM))

    attn = pl.pallas_call(
        _attn_body,
        grid=(NQ,),
        in_specs=[
            pl.BlockSpec((QB, NPROJ), lambda i: (i, 0)),
            pl.BlockSpec((S, 2 * G * DK), lambda i: (0, 2)),
        ],
        out_specs=pl.BlockSpec((QB, H * DV), lambda i: (i, 0)),
        out_shape=jax.ShapeDtypeStruct((S, H * DV), f32),
        scratch_shapes=[
            pltpu.VMEM((S, 2 * G * DK), bf16),
            pltpu.VMEM((G * 128, DK), f32),
            pltpu.VMEM((G * 128, DV), f32),
            pltpu.VMEM((NKT, HPG * QB, KB), bf16),
        ],
    )(qkvg, qkvg)

    x2, hbf = pl.pallas_call(
        _c1_body,
        grid=(NQ,),
        in_specs=[
            pl.BlockSpec((QB, H * DV), lambda i: (i, 0)),
            pl.BlockSpec((H * DV, DIM), lambda i: (0, 0)),
            pl.BlockSpec((QB, DIM), lambda i: (i, 0)),
            pl.BlockSpec((1, DIM), lambda i: (0, 0)),
        ],
        out_specs=[pl.BlockSpec((QB, DIM), lambda i: (i, 0)),
                   pl.BlockSpec((QB, DIM), lambda i: (i, 0))],
        out_shape=[jax.ShapeDtypeStruct((S, DIM), f32),
                   jax.ShapeDtypeStruct((S, DIM), bf16)],
    )(attn, Wo.astype(bf16), xs, norm2_w.reshape(1, DIM))

    NC2 = 16
    CH = HMULT * DIM // NC2
    y = pl.pallas_call(
        _c2_body,
        grid=(NC2,),
        in_specs=[
            pl.BlockSpec((S, DIM), lambda j: (0, 0)),
            pl.BlockSpec((S, DIM), lambda j: (0, 0)),
            pl.BlockSpec((DIM, CH), lambda j: (0, j)),
            pl.BlockSpec((CH, DIM), lambda j: (j, 0)),
        ],
        out_specs=pl.BlockSpec((S, DIM), lambda j: (0, 0)),
        out_shape=jax.ShapeDtypeStruct((S, DIM), f32),
    )(hbf, x2, W1.astype(bf16), W2.astype(bf16))

    return y.reshape(B, S, DIM)


# speedup vs baseline: 1.5774x; 1.0718x over previous
import functools

import jax
import jax.numpy as jnp
from jax import lax
from jax.experimental import pallas as pl
from jax.experimental.pallas import tpu as pltpu

B, S, DIM = 1, 2048, 2048
H, G, DK, DV = 16, 4, 128, 128
HPG = H // G
L, D, LSEL, NSEL, W = 32, 16, 64, 16, 512
NCMP = (S - L) // D + 1
NSB = S // LSEL
HMULT = 4

QB = 256
KB = 256
NQ = S // QB
NKT = S // KB
NEG = -1e9
SCALE = 1.0 / (DK ** 0.5)
NPROJ = H * DK + G * DK + G * DV + 128
GOFF = H * DK + G * DK + G * DV
HI = jax.lax.Precision.HIGHEST

f32 = jnp.float32
bf16 = jnp.bfloat16


def _iota(shape, dim):
    return lax.broadcasted_iota(jnp.int32, shape, dim)


def _proj_body(x_ref, w_ref, nw_ref, o_ref):
    xb = x_ref[...]
    ms = jnp.mean(xb * xb, axis=-1, keepdims=True)
    xn = xb * lax.rsqrt(ms + 1e-6) * nw_ref[...]
    o_ref[...] = jnp.dot(xn.astype(bf16), w_ref[...],
                         preferred_element_type=f32)


def _attn_body(qg_ref, kv_ref, o_ref, kvbf_ref, kcmp_ref, vcmp_ref, selm_ref,
               knm_ref):
    qi = pl.program_id(0)
    t0 = qi * QB
    t = t0 + _iota((QB, 1), 0)

    @pl.when(qi == 0)
    def _init():
        kvbf_ref[...] = kv_ref[...].astype(bf16)
        ci = _iota((128, S), 0)
        si = _iota((128, S), 1)
        P = jnp.where((si >= ci * D) & (si < ci * D + L), f32(1.0 / L),
                      f32(0.0))
        for g in range(G):
            kf = kv_ref[:, g * DK:(g + 1) * DK]
            vf = kv_ref[:, G * DK + g * DV:G * DK + (g + 1) * DV]
            kcmp_ref[g * 128:(g + 1) * 128, :] = jnp.dot(
                P, kf, preferred_element_type=f32, precision=HI)
            vcmp_ref[g * 128:(g + 1) * 128, :] = jnp.dot(
                P, vf, preferred_element_type=f32, precision=HI)
            kn = jnp.sum(kf * kf, axis=-1, keepdims=True)
            knm_ref[g:g + 1, 0:1] = jnp.max(kn, axis=0, keepdims=True)

    QH = HPG * QB
    t4 = t0 + _iota((QH, 1), 0) % QB
    ci4 = _iota((QH, 128), 1)
    vis4 = ((ci4 * D + L - 1) <= t4) & (ci4 < NCMP)
    visf4 = vis4.astype(f32)

    Mseg = jnp.where((_iota((128, 128), 0) // 4 == _iota((128, 128), 1))
                     & (_iota((128, 128), 0) < NCMP), f32(1.0), f32(0.0))

    jlane = _iota((QB, 128), 1)

    for g in range(G):
        kcmp = kcmp_ref[g * 128:(g + 1) * 128, :]
        vcmp = vcmp_ref[g * 128:(g + 1) * 128, :]

        q4 = jnp.concatenate(
            [qg_ref[:, (g * HPG + h) * DK:(g * HPG + h + 1) * DK]
             for h in range(HPG)], axis=0)
        q4bf = (q4 * SCALE).astype(bf16)

        sc = lax.dot_general(q4, kcmp, (((1,), (1,)), ((), ())),
                             precision=HI,
                             preferred_element_type=f32) * SCALE
        sc = jnp.where(vis4, sc, NEG)
        m = jnp.max(sc, axis=-1, keepdims=True)
        p = jnp.exp(sc - m)
        p = p / jnp.sum(p, axis=-1, keepdims=True)
        p = p * visf4
        out_cmp4 = jnp.dot(p.astype(bf16), vcmp.astype(bf16),
                           preferred_element_type=f32)
        p_imp = (p[0:QB] + p[QB:2 * QB] + p[2 * QB:3 * QB] + p[3 * QB:4 * QB])

        selr = jnp.dot(p_imp, Mseg, precision=HI, preferred_element_type=f32)
        allowed = (jlane * LSEL <= t) & (jlane < NSB)
        force = (jlane == 0) | (jlane == t // LSEL)
        s = jnp.where(allowed, selr + 1e9 * force.astype(f32), NEG)
        cnt = jnp.zeros((QB, 128), jnp.int32)
        for jj in range(NSB):
            cnt = cnt + (s[:, jj:jj + 1] > s).astype(jnp.int32)
        mask_blk = ((cnt < NSEL) & (jlane < NSB)).astype(f32)

        def _mk(kb, _):
            rep = jnp.where(
                ((kb * KB + _iota((128, KB), 1)) // LSEL) == _iota((128, KB), 0),
                f32(1.0), f32(0.0))
            mt = jnp.dot(mask_blk.astype(bf16), rep.astype(bf16),
                         preferred_element_type=f32)
            bias = ((mt - 1.0) * 1e9).astype(bf16)
            for h in range(HPG):
                selm_ref[kb, h * QB:(h + 1) * QB, :] = bias
            return 0

        lax.fori_loop(0, qi + 1, _mk, 0)

        qn = jnp.sum(q4 * q4, axis=-1, keepdims=True)
        mb = jnp.sqrt(qn) * jnp.sqrt(knm_ref[g:g + 1, 0:1]) * SCALE

        onespad = jnp.where(_iota((KB, 8), 1) == 0, 1.0, 0.0).astype(bf16)

        def _kv(kb):
            kblk = kvbf_ref[pl.ds(kb * KB, KB), g * DK:(g + 1) * DK]
            vblk = kvbf_ref[pl.ds(kb * KB, KB),
                            G * DK + g * DV:G * DK + (g + 1) * DV]
            return kblk, jnp.concatenate([vblk, onespad], axis=1)

        def _tile(kb, ae):
            kblk, vext = _kv(kb)
            qk = lax.dot_general(q4bf, kblk, (((1,), (1,)), ((), ())),
                                 preferred_element_type=f32)
            e = jnp.exp(qk + selm_ref[kb] - mb)
            return ae + jnp.dot(e.astype(bf16), vext,
                                preferred_element_type=f32)

        ae_s = lax.fori_loop(0, jnp.maximum(qi - 2, 0), _tile,
                             jnp.zeros((QH, DV + 8), f32))
        ae_w = jnp.zeros((QH, DV + 8), f32)

        for r in range(3):
            kbr = qi - 2 + r
            kbc = jnp.maximum(kbr, 0)
            kblk, vext = _kv(kbc)
            qk = lax.dot_general(q4bf, kblk, (((1,), (1,)), ((), ())),
                                 preferred_element_type=f32)
            if r < 2:
                qk = jnp.where(kbr >= 0, qk, NEG)
            else:
                pcol = kbc * KB + _iota((QH, KB), 1)
                qk = jnp.where(pcol <= t4, qk, NEG)
            es = jnp.exp(qk + selm_ref[kbc] - mb)
            ae_s = ae_s + jnp.dot(es.astype(bf16), vext,
                                  preferred_element_type=f32)
            if r == 0:
                pcol = kbc * KB + _iota((QH, KB), 1)
                sw = jnp.where(pcol > t4 - W, qk, NEG)
            else:
                sw = qk
            ew = jnp.exp(sw - mb)
            ae_w = ae_w + jnp.dot(ew.astype(bf16), vext,
                                  preferred_element_type=f32)

        out_sel4 = ae_s[:, :DV] / ae_s[:, DV:DV + 1]
        out_win4 = ae_w[:, :DV] / ae_w[:, DV:DV + 1]

        for h in range(HPG):
            hg = g * HPG + h
            row = slice(h * QB, (h + 1) * QB)

            def _gate(e):
                gcol = qg_ref[:, GOFF + hg * 3 + e:GOFF + hg * 3 + e + 1]
                return 1.0 / (1.0 + jnp.exp(-gcol))

            out_h = (_gate(0) * out_cmp4[row] + _gate(1) * out_sel4[row]
                     + _gate(2) * out_win4[row])
            o_ref[:, hg * DV:(hg + 1) * DV] = out_h


def _c1_body(a_ref, wo_ref, x_ref, nw_ref, x2_ref, h_ref):
    x2 = jnp.dot(a_ref[...].astype(bf16), wo_ref[...],
                 preferred_element_type=f32) + x_ref[...]
    x2_ref[...] = x2
    ms = jnp.mean(x2 * x2, axis=-1, keepdims=True)
    h_ref[...] = (x2 * lax.rsqrt(ms + 1e-6) * nw_ref[...]).astype(bf16)


def _c2_body(h_ref, x2_ref, w1_ref, w2_ref, o_ref):
    j = pl.program_id(0)
    for mi in range(8):
        sl = slice(mi * 256, (mi + 1) * 256)
        a = jnp.dot(h_ref[sl, :], w1_ref[...], preferred_element_type=f32)
        a = a * (1.0 / (1.0 + jnp.exp(-a)))
        contrib = jnp.dot(a.astype(bf16), w2_ref[...],
                          preferred_element_type=f32)

        @pl.when(j == 0)
        def _():
            o_ref[sl, :] = x2_ref[sl, :] + contrib

        @pl.when(j > 0)
        def _():
            o_ref[sl, :] = o_ref[sl, :] + contrib


def kernel(x, norm1_w, Wq, Wk, Wv, Wg, Wo, norm2_w, W1, W2):
    xs = x.reshape(S, DIM)
    Wall = jnp.concatenate(
        [Wq, Wk, Wv, jnp.pad(Wg, ((0, 0), (0, 128 - H * 3)))],
        axis=1).astype(bf16)

    qkvg = pl.pallas_call(
        _proj_body,
        grid=(NQ,),
        in_specs=[
            pl.BlockSpec((QB, DIM), lambda i: (i, 0)),
            pl.BlockSpec((DIM, NPROJ), lambda i: (0, 0)),
            pl.BlockSpec((1, DIM), lambda i: (0, 0)),
        ],
        out_specs=pl.BlockSpec((QB, NPROJ), lambda i: (i, 0)),
        out_shape=jax.ShapeDtypeStruct((S, NPROJ), f32),
    )(xs, Wall, norm1_w.reshape(1, DIM))

    attn = pl.pallas_call(
        _attn_body,
        grid=(NQ,),
        in_specs=[
            pl.BlockSpec((QB, NPROJ), lambda i: (i, 0)),
            pl.BlockSpec((S, 2 * G * DK), lambda i: (0, 2)),
        ],
        out_specs=pl.BlockSpec((QB, H * DV), lambda i: (i, 0)),
        out_shape=jax.ShapeDtypeStruct((S, H * DV), f32),
        scratch_shapes=[
            pltpu.VMEM((S, 2 * G * DK), bf16),
            pltpu.VMEM((G * 128, DK), f32),
            pltpu.VMEM((G * 128, DV), f32),
            pltpu.VMEM((NKT, HPG * QB, KB), bf16),
            pltpu.VMEM((8, 128), f32),
        ],
    )(qkvg, qkvg)

    x2, hbf = pl.pallas_call(
        _c1_body,
        grid=(NQ,),
        in_specs=[
            pl.BlockSpec((QB, H * DV), lambda i: (i, 0)),
            pl.BlockSpec((H * DV, DIM), lambda i: (0, 0)),
            pl.BlockSpec((QB, DIM), lambda i: (i, 0)),
            pl.BlockSpec((1, DIM), lambda i: (0, 0)),
        ],
        out_specs=[pl.BlockSpec((QB, DIM), lambda i: (i, 0)),
                   pl.BlockSpec((QB, DIM), lambda i: (i, 0))],
        out_shape=[jax.ShapeDtypeStruct((S, DIM), f32),
                   jax.ShapeDtypeStruct((S, DIM), bf16)],
    )(attn, Wo.astype(bf16), xs, norm2_w.reshape(1, DIM))

    NC2 = 16
    CH = HMULT * DIM // NC2
    y = pl.pallas_call(
        _c2_body,
        grid=(NC2,),
        in_specs=[
            pl.BlockSpec((S, DIM), lambda j: (0, 0)),
            pl.BlockSpec((S, DIM), lambda j: (0, 0)),
            pl.BlockSpec((DIM, CH), lambda j: (0, j)),
            pl.BlockSpec((CH, DIM), lambda j: (j, 0)),
        ],
        out_specs=pl.BlockSpec((S, DIM), lambda j: (0, 0)),
        out_shape=jax.ShapeDtypeStruct((S, DIM), f32),
    )(hbf, x2, W1.astype(bf16), W2.astype(bf16))

    return y.reshape(B, S, DIM)


# speedup vs baseline: 1.6032x; 1.0164x over previous
import functools

import jax
import jax.numpy as jnp
from jax import lax
from jax.experimental import pallas as pl
from jax.experimental.pallas import tpu as pltpu

B, S, DIM = 1, 2048, 2048
H, G, DK, DV = 16, 4, 128, 128
HPG = H // G
L, D, LSEL, NSEL, W = 32, 16, 64, 16, 512
NCMP = (S - L) // D + 1
NSB = S // LSEL
HMULT = 4

QB = 256
KB = 256
NQ = S // QB
NKT = S // KB
NEG = -1e9
SCALE = 1.0 / (DK ** 0.5)
NPROJ = H * DK + G * DK + G * DV + 128
GOFF = H * DK + G * DK + G * DV
HI = jax.lax.Precision.HIGHEST

f32 = jnp.float32
bf16 = jnp.bfloat16


def _iota(shape, dim):
    return lax.broadcasted_iota(jnp.int32, shape, dim)


def _proj_body(x_ref, w_ref, nw_ref, o_ref):
    xb = x_ref[...]
    ms = jnp.mean(xb * xb, axis=-1, keepdims=True)
    xn = xb * lax.rsqrt(ms + 1e-6) * nw_ref[...]
    o_ref[...] = jnp.dot(xn.astype(bf16), w_ref[...],
                         preferred_element_type=f32)


def _attn_body(qg_ref, kv_ref, o_ref, kvbf_ref, kcmp_ref, vcmp_ref, selm_ref,
               knm_ref):
    qi = pl.program_id(0)
    t0 = qi * QB
    t = t0 + _iota((QB, 1), 0)

    @pl.when(qi == 0)
    def _init():
        kvbf_ref[...] = kv_ref[...].astype(bf16)
        ci = _iota((128, S), 0)
        si = _iota((128, S), 1)
        P = jnp.where((si >= ci * D) & (si < ci * D + L), f32(1.0 / L),
                      f32(0.0))
        for g in range(G):
            kf = kv_ref[:, g * DK:(g + 1) * DK]
            vf = kv_ref[:, G * DK + g * DV:G * DK + (g + 1) * DV]
            kcmp_ref[g * 128:(g + 1) * 128, :] = jnp.dot(
                P, kf, preferred_element_type=f32, precision=HI)
            vcmp_ref[g * 128:(g + 1) * 128, :] = jnp.dot(
                P, vf, preferred_element_type=f32, precision=HI)
            kn = jnp.sum(kf * kf, axis=-1, keepdims=True)
            knm_ref[g:g + 1, 0:1] = jnp.max(kn, axis=0, keepdims=True)

    QH = HPG * QB
    t4 = t0 + _iota((QH, 1), 0) % QB
    ci4 = _iota((QH, 128), 1)
    vis4 = ((ci4 * D + L - 1) <= t4) & (ci4 < NCMP)
    visf4 = vis4.astype(f32)

    Mseg = jnp.where((_iota((128, 128), 0) // 4 == _iota((128, 128), 1))
                     & (_iota((128, 128), 0) < NCMP), f32(1.0), f32(0.0))

    jlane = _iota((QB, 128), 1)

    for g in range(G):
        kcmp = kcmp_ref[g * 128:(g + 1) * 128, :]
        vcmp = vcmp_ref[g * 128:(g + 1) * 128, :]

        q4 = jnp.concatenate(
            [qg_ref[:, (g * HPG + h) * DK:(g * HPG + h + 1) * DK]
             for h in range(HPG)], axis=0)
        q4bf = (q4 * SCALE).astype(bf16)

        qn = jnp.sum(q4 * q4, axis=-1, keepdims=True)
        mb = jnp.sqrt(qn) * jnp.sqrt(knm_ref[g:g + 1, 0:1]) * SCALE

        sc = lax.dot_general(q4, kcmp, (((1,), (1,)), ((), ())),
                             precision=HI,
                             preferred_element_type=f32) * SCALE
        sc = jnp.where(vis4, sc, NEG)
        p = jnp.exp(sc - mb)
        p = p / (jnp.sum(p, axis=-1, keepdims=True) + 1e-30)
        p = p * visf4
        out_cmp4 = jnp.dot(p.astype(bf16), vcmp.astype(bf16),
                           preferred_element_type=f32)
        p_imp = (p[0:QB] + p[QB:2 * QB] + p[2 * QB:3 * QB] + p[3 * QB:4 * QB])

        selr = jnp.dot(p_imp, Mseg, precision=HI, preferred_element_type=f32)
        allowed = (jlane * LSEL <= t) & (jlane < NSB)
        force = (jlane == 0) | (jlane == t // LSEL)
        s = jnp.where(allowed, selr + 1e9 * force.astype(f32), NEG)
        cnt = jnp.zeros((QB, 128), jnp.int32)
        for jj in range(NSB):
            cnt = cnt + (s[:, jj:jj + 1] > s).astype(jnp.int32)
        mask_blk = ((cnt < NSEL) & (jlane < NSB)).astype(f32)

        def _mk(kb, _):
            rep = jnp.where(
                ((kb * KB + _iota((128, KB), 1)) // LSEL) == _iota((128, KB), 0),
                f32(1.0), f32(0.0))
            mt = jnp.dot(mask_blk.astype(bf16), rep.astype(bf16),
                         preferred_element_type=f32)
            bias = ((mt - 1.0) * 1e9).astype(bf16)
            for h in range(HPG):
                selm_ref[kb, h * QB:(h + 1) * QB, :] = bias
            return 0

        lax.fori_loop(0, qi + 1, _mk, 0)

        onespad = jnp.where(_iota((KB, 8), 1) == 0, 1.0, 0.0).astype(bf16)

        def _kv(kb):
            kblk = kvbf_ref[pl.ds(kb * KB, KB), g * DK:(g + 1) * DK]
            vblk = kvbf_ref[pl.ds(kb * KB, KB),
                            G * DK + g * DV:G * DK + (g + 1) * DV]
            return kblk, jnp.concatenate([vblk, onespad], axis=1)

        def _tile(kb, ae):
            kblk, vext = _kv(kb)
            qk = lax.dot_general(q4bf, kblk, (((1,), (1,)), ((), ())),
                                 preferred_element_type=f32)
            e = jnp.exp(qk + selm_ref[kb] - mb)
            return ae + jnp.dot(e.astype(bf16), vext,
                                preferred_element_type=f32)

        ae_s = lax.fori_loop(0, jnp.maximum(qi - 2, 0), _tile,
                             jnp.zeros((QH, DV + 8), f32))
        ae_w = jnp.zeros((QH, DV + 8), f32)

        for r in range(3):
            kbr = qi - 2 + r
            kbc = jnp.maximum(kbr, 0)
            kblk, vext = _kv(kbc)
            qk = lax.dot_general(q4bf, kblk, (((1,), (1,)), ((), ())),
                                 preferred_element_type=f32)
            if r < 2:
                qk = jnp.where(kbr >= 0, qk, NEG)
            else:
                pcol = kbc * KB + _iota((QH, KB), 1)
                qk = jnp.where(pcol <= t4, qk, NEG)
            es = jnp.exp(qk + selm_ref[kbc] - mb)
            ae_s = ae_s + jnp.dot(es.astype(bf16), vext,
                                  preferred_element_type=f32)
            if r == 0:
                pcol = kbc * KB + _iota((QH, KB), 1)
                sw = jnp.where(pcol > t4 - W, qk, NEG)
            else:
                sw = qk
            ew = jnp.exp(sw - mb)
            ae_w = ae_w + jnp.dot(ew.astype(bf16), vext,
                                  preferred_element_type=f32)

        out_sel4 = ae_s[:, :DV] / ae_s[:, DV:DV + 1]
        out_win4 = ae_w[:, :DV] / ae_w[:, DV:DV + 1]

        for h in range(HPG):
            hg = g * HPG + h
            row = slice(h * QB, (h + 1) * QB)

            def _gate(e):
                gcol = qg_ref[:, GOFF + hg * 3 + e:GOFF + hg * 3 + e + 1]
                return 1.0 / (1.0 + jnp.exp(-gcol))

            out_h = (_gate(0) * out_cmp4[row] + _gate(1) * out_sel4[row]
                     + _gate(2) * out_win4[row])
            o_ref[:, hg * DV:(hg + 1) * DV] = out_h


def _c1_body(a_ref, wo_ref, x_ref, nw_ref, x2_ref, h_ref):
    x2 = jnp.dot(a_ref[...].astype(bf16), wo_ref[...],
                 preferred_element_type=f32) + x_ref[...]
    x2_ref[...] = x2
    ms = jnp.mean(x2 * x2, axis=-1, keepdims=True)
    h_ref[...] = (x2 * lax.rsqrt(ms + 1e-6) * nw_ref[...]).astype(bf16)


def _c2_body(h_ref, x2_ref, w1_ref, w2_ref, o_ref):
    j = pl.program_id(0)
    for mi in range(8):
        sl = slice(mi * 256, (mi + 1) * 256)
        a = jnp.dot(h_ref[sl, :], w1_ref[...], preferred_element_type=f32)
        a = a * (1.0 / (1.0 + jnp.exp(-a)))
        contrib = jnp.dot(a.astype(bf16), w2_ref[...],
                          preferred_element_type=f32)

        @pl.when(j == 0)
        def _():
            o_ref[sl, :] = x2_ref[sl, :] + contrib

        @pl.when(j > 0)
        def _():
            o_ref[sl, :] = o_ref[sl, :] + contrib


def kernel(x, norm1_w, Wq, Wk, Wv, Wg, Wo, norm2_w, W1, W2):
    xs = x.reshape(S, DIM)
    Wall = jnp.concatenate(
        [Wq, Wk, Wv, jnp.pad(Wg, ((0, 0), (0, 128 - H * 3)))],
        axis=1).astype(bf16)

    qkvg = pl.pallas_call(
        _proj_body,
        grid=(NQ,),
        in_specs=[
            pl.BlockSpec((QB, DIM), lambda i: (i, 0)),
            pl.BlockSpec((DIM, NPROJ), lambda i: (0, 0)),
            pl.BlockSpec((1, DIM), lambda i: (0, 0)),
        ],
        out_specs=pl.BlockSpec((QB, NPROJ), lambda i: (i, 0)),
        out_shape=jax.ShapeDtypeStruct((S, NPROJ), f32),
    )(xs, Wall, norm1_w.reshape(1, DIM))

    attn = pl.pallas_call(
        _attn_body,
        grid=(NQ,),
        in_specs=[
            pl.BlockSpec((QB, NPROJ), lambda i: (i, 0)),
            pl.BlockSpec((S, 2 * G * DK), lambda i: (0, 2)),
        ],
        out_specs=pl.BlockSpec((QB, H * DV), lambda i: (i, 0)),
        out_shape=jax.ShapeDtypeStruct((S, H * DV), f32),
        scratch_shapes=[
            pltpu.VMEM((S, 2 * G * DK), bf16),
            pltpu.VMEM((G * 128, DK), f32),
            pltpu.VMEM((G * 128, DV), f32),
            pltpu.VMEM((NKT, HPG * QB, KB), bf16),
            pltpu.VMEM((8, 128), f32),
        ],
    )(qkvg, qkvg)

    x2, hbf = pl.pallas_call(
        _c1_body,
        grid=(NQ,),
        in_specs=[
            pl.BlockSpec((QB, H * DV), lambda i: (i, 0)),
            pl.BlockSpec((H * DV, DIM), lambda i: (0, 0)),
            pl.BlockSpec((QB, DIM), lambda i: (i, 0)),
            pl.BlockSpec((1, DIM), lambda i: (0, 0)),
        ],
        out_specs=[pl.BlockSpec((QB, DIM), lambda i: (i, 0)),
                   pl.BlockSpec((QB, DIM), lambda i: (i, 0))],
        out_shape=[jax.ShapeDtypeStruct((S, DIM), f32),
                   jax.ShapeDtypeStruct((S, DIM), bf16)],
    )(attn, Wo.astype(bf16), xs, norm2_w.reshape(1, DIM))

    NC2 = 16
    CH = HMULT * DIM // NC2
    y = pl.pallas_call(
        _c2_body,
        grid=(NC2,),
        in_specs=[
            pl.BlockSpec((S, DIM), lambda j: (0, 0)),
            pl.BlockSpec((S, DIM), lambda j: (0, 0)),
            pl.BlockSpec((DIM, CH), lambda j: (0, j)),
            pl.BlockSpec((CH, DIM), lambda j: (j, 0)),
        ],
        out_specs=pl.BlockSpec((S, DIM), lambda j: (0, 0)),
        out_shape=jax.ShapeDtypeStruct((S, DIM), f32),
    )(hbf, x2, W1.astype(bf16), W2.astype(bf16))

    return y.reshape(B, S, DIM)


# speedup vs baseline: 1.7291x; 1.0785x over previous
import functools

import jax
import jax.numpy as jnp
from jax import lax
from jax.experimental import pallas as pl
from jax.experimental.pallas import tpu as pltpu

B, S, DIM = 1, 2048, 2048
H, G, DK, DV = 16, 4, 128, 128
HPG = H // G
L, D, LSEL, NSEL, W = 32, 16, 64, 16, 512
NCMP = (S - L) // D + 1
NSB = S // LSEL
HMULT = 4

QB = 256
KB = 256
NQ = S // QB
NKT = S // KB
NEG = -1e9
SCALE = 1.0 / (DK ** 0.5)
NPROJ = H * DK + G * DK + G * DV + 128
GOFF = H * DK + G * DK + G * DV
HI = jax.lax.Precision.HIGHEST

f32 = jnp.float32
bf16 = jnp.bfloat16


def _iota(shape, dim):
    return lax.broadcasted_iota(jnp.int32, shape, dim)


def _proj_body(x_ref, w_ref, nw_ref, o_ref):
    xb = x_ref[...]
    ms = jnp.mean(xb * xb, axis=-1, keepdims=True)
    xn = xb * lax.rsqrt(ms + 1e-6) * nw_ref[...]
    o_ref[...] = jnp.dot(xn.astype(bf16), w_ref[...],
                         preferred_element_type=f32)


def _attn_body(qg_ref, kv_ref, o_ref, kvbf_ref, kcmp_ref, vcmp_ref, selm_ref,
               knm_ref):
    qi = pl.program_id(0)
    t0 = qi * QB
    t = t0 + _iota((QB, 1), 0)

    @pl.when(qi == 0)
    def _init():
        kvbf_ref[...] = kv_ref[...].astype(bf16)
        ci = _iota((128, S), 0)
        si = _iota((128, S), 1)
        P = jnp.where((si >= ci * D) & (si < ci * D + L), f32(1.0 / L),
                      f32(0.0))
        for g in range(G):
            kf = kv_ref[:, g * DK:(g + 1) * DK]
            vf = kv_ref[:, G * DK + g * DV:G * DK + (g + 1) * DV]
            kcmp_ref[g * 128:(g + 1) * 128, :] = jnp.dot(
                P, kf, preferred_element_type=f32, precision=HI)
            vcmp_ref[g * 128:(g + 1) * 128, :] = jnp.dot(
                P, vf, preferred_element_type=f32, precision=HI)
            kn = jnp.sum(kf * kf, axis=-1, keepdims=True)
            knm_ref[g:g + 1, 0:1] = jnp.max(kn, axis=0, keepdims=True)

    QH = HPG * QB
    t4 = t0 + _iota((QH, 1), 0) % QB
    ci4 = _iota((QH, 128), 1)
    vis4 = ((ci4 * D + L - 1) <= t4) & (ci4 < NCMP)
    visf4 = vis4.astype(f32)

    Mseg = jnp.where((_iota((128, 128), 0) // 4 == _iota((128, 128), 1))
                     & (_iota((128, 128), 0) < NCMP), f32(1.0), f32(0.0))

    jlane = _iota((QB, 128), 1)

    for g in range(G):
        kcmp = kcmp_ref[g * 128:(g + 1) * 128, :]
        vcmp = vcmp_ref[g * 128:(g + 1) * 128, :]

        q4 = jnp.concatenate(
            [qg_ref[:, (g * HPG + h) * DK:(g * HPG + h + 1) * DK]
             for h in range(HPG)], axis=0)
        q4bf = (q4 * SCALE).astype(bf16)

        qn = jnp.sum(q4 * q4, axis=-1, keepdims=True)
        mb = jnp.sqrt(qn) * jnp.sqrt(knm_ref[g:g + 1, 0:1]) * SCALE

        sc = lax.dot_general(q4, kcmp, (((1,), (1,)), ((), ())),
                             precision=HI,
                             preferred_element_type=f32) * SCALE
        sc = jnp.where(vis4, sc, NEG)
        p = jnp.exp(sc - mb)
        p = p / (jnp.sum(p, axis=-1, keepdims=True) + 1e-30)
        p = p * visf4
        out_cmp4 = jnp.dot(p.astype(bf16), vcmp.astype(bf16),
                           preferred_element_type=f32)
        p_imp = (p[0:QB] + p[QB:2 * QB] + p[2 * QB:3 * QB] + p[3 * QB:4 * QB])

        selr = jnp.dot(p_imp, Mseg, precision=HI, preferred_element_type=f32)
        allowed = (jlane * LSEL <= t) & (jlane < NSB)
        force = (jlane == 0) | (jlane == t // LSEL)
        s = jnp.where(allowed, selr + 1e9 * force.astype(f32), NEG)
        cnt = jnp.zeros((QB, 128), jnp.int32)
        for jj in range(NSB):
            cnt = cnt + (s[:, jj:jj + 1] > s).astype(jnp.int32)
        mask_blk = ((cnt < NSEL) & (jlane < NSB)).astype(f32)

        def _mk(kb, _):
            rep = jnp.where(
                ((kb * KB + _iota((128, KB), 1)) // LSEL) == _iota((128, KB), 0),
                f32(1.0), f32(0.0))
            mt = jnp.dot(mask_blk.astype(bf16), rep.astype(bf16),
                         preferred_element_type=f32)
            bias = ((mt - 1.0) * 1e9).astype(bf16)
            for h in range(HPG):
                selm_ref[kb, h * QB:(h + 1) * QB, :] = bias
            return 0

        lax.fori_loop(0, qi + 1, _mk, 0)

        onespad = jnp.where(_iota((KB, 8), 1) == 0, 1.0, 0.0).astype(bf16)

        def _kv(kb):
            kblk = kvbf_ref[pl.ds(kb * KB, KB), g * DK:(g + 1) * DK]
            vblk = kvbf_ref[pl.ds(kb * KB, KB),
                            G * DK + g * DV:G * DK + (g + 1) * DV]
            return kblk, jnp.concatenate([vblk, onespad], axis=1)

        def _tile(kb, ae):
            kblk, vext = _kv(kb)
            qk = lax.dot_general(q4bf, kblk, (((1,), (1,)), ((), ())),
                                 preferred_element_type=f32)
            e = jnp.exp(qk + selm_ref[kb] - mb)
            return ae + jnp.dot(e.astype(bf16), vext,
                                preferred_element_type=f32)

        ae_s = lax.fori_loop(0, jnp.maximum(qi - 2, 0), _tile,
                             jnp.zeros((QH, DV + 8), f32))
        ae_w = jnp.zeros((QH, DV + 8), f32)

        for r in range(3):
            kbr = qi - 2 + r
            kbc = jnp.maximum(kbr, 0)
            kblk, vext = _kv(kbc)
            qk = lax.dot_general(q4bf, kblk, (((1,), (1,)), ((), ())),
                                 preferred_element_type=f32)
            if r < 2:
                qk = jnp.where(kbr >= 0, qk, NEG)
            else:
                pcol = kbc * KB + _iota((QH, KB), 1)
                qk = jnp.where(pcol <= t4, qk, NEG)
            es = jnp.exp(qk + selm_ref[kbc] - mb)
            ae_s = ae_s + jnp.dot(es.astype(bf16), vext,
                                  preferred_element_type=f32)
            if r == 0:
                pcol = kbc * KB + _iota((QH, KB), 1)
                sw = jnp.where(pcol > t4 - W, qk, NEG)
            else:
                sw = qk
            ew = jnp.exp(sw - mb)
            ae_w = ae_w + jnp.dot(ew.astype(bf16), vext,
                                  preferred_element_type=f32)

        out_sel4 = ae_s[:, :DV] / ae_s[:, DV:DV + 1]
        out_win4 = ae_w[:, :DV] / ae_w[:, DV:DV + 1]

        for h in range(HPG):
            hg = g * HPG + h
            row = slice(h * QB, (h + 1) * QB)

            def _gate(e):
                gcol = qg_ref[:, GOFF + hg * 3 + e:GOFF + hg * 3 + e + 1]
                return 1.0 / (1.0 + jnp.exp(-gcol))

            out_h = (_gate(0) * out_cmp4[row] + _gate(1) * out_sel4[row]
                     + _gate(2) * out_win4[row])
            o_ref[:, hg * DV:(hg + 1) * DV] = out_h


def _c1_body(a_ref, wo_ref, x_ref, nw_ref, x2_ref, h_ref):
    x2 = jnp.dot(a_ref[...].astype(bf16), wo_ref[...],
                 preferred_element_type=f32) + x_ref[...]
    x2_ref[...] = x2
    ms = jnp.mean(x2 * x2, axis=-1, keepdims=True)
    h_ref[...] = (x2 * lax.rsqrt(ms + 1e-6) * nw_ref[...]).astype(bf16)


def _c2_body(h_ref, x2_ref, w1_ref, w2_ref, o_ref):
    j = pl.program_id(0)
    for mi in range(8):
        sl = slice(mi * 256, (mi + 1) * 256)
        a = jnp.dot(h_ref[sl, :], w1_ref[...], preferred_element_type=f32)
        a = a * (1.0 / (1.0 + jnp.exp(-a)))
        contrib = jnp.dot(a.astype(bf16), w2_ref[...],
                          preferred_element_type=f32)

        @pl.when(j == 0)
        def _():
            o_ref[sl, :] = x2_ref[sl, :] + contrib

        @pl.when(j > 0)
        def _():
            o_ref[sl, :] = o_ref[sl, :] + contrib


def kernel(x, norm1_w, Wq, Wk, Wv, Wg, Wo, norm2_w, W1, W2):
    xs = x.reshape(S, DIM)
    Wall = jnp.concatenate(
        [Wq, Wk, Wv, jnp.pad(Wg, ((0, 0), (0, 128 - H * 3)))],
        axis=1).astype(bf16)

    qkvg = pl.pallas_call(
        _proj_body,
        grid=(NQ,),
        in_specs=[
            pl.BlockSpec((QB, DIM), lambda i: (i, 0)),
            pl.BlockSpec((DIM, NPROJ), lambda i: (0, 0)),
            pl.BlockSpec((1, DIM), lambda i: (0, 0)),
        ],
        out_specs=pl.BlockSpec((QB, NPROJ), lambda i: (i, 0)),
        out_shape=jax.ShapeDtypeStruct((S, NPROJ), f32),
    )(xs, Wall, norm1_w.reshape(1, DIM))

    attn = pl.pallas_call(
        _attn_body,
        grid=(NQ,),
        in_specs=[
            pl.BlockSpec((QB, NPROJ), lambda i: (i, 0)),
            pl.BlockSpec((S, 2 * G * DK), lambda i: (0, 2)),
        ],
        out_specs=pl.BlockSpec((QB, H * DV), lambda i: (i, 0)),
        out_shape=jax.ShapeDtypeStruct((S, H * DV), f32),
        scratch_shapes=[
            pltpu.VMEM((S, 2 * G * DK), bf16),
            pltpu.VMEM((G * 128, DK), f32),
            pltpu.VMEM((G * 128, DV), f32),
            pltpu.VMEM((NKT, HPG * QB, KB), bf16),
            pltpu.VMEM((8, 128), f32),
        ],
    )(qkvg, qkvg)

    x2, hbf = pl.pallas_call(
        _c1_body,
        grid=(NQ,),
        in_specs=[
            pl.BlockSpec((QB, H * DV), lambda i: (i, 0)),
            pl.BlockSpec((H * DV, DIM), lambda i: (0, 0)),
            pl.BlockSpec((QB, DIM), lambda i: (i, 0)),
            pl.BlockSpec((1, DIM), lambda i: (0, 0)),
        ],
        out_specs=[pl.BlockSpec((QB, DIM), lambda i: (i, 0)),
                   pl.BlockSpec((QB, DIM), lambda i: (i, 0))],
        out_shape=[jax.ShapeDtypeStruct((S, DIM), f32),
                   jax.ShapeDtypeStruct((S, DIM), bf16)],
    )(attn, Wo.astype(bf16), xs, norm2_w.reshape(1, DIM))

    NC2 = 8
    CH = HMULT * DIM // NC2
    y = pl.pallas_call(
        _c2_body,
        grid=(NC2,),
        in_specs=[
            pl.BlockSpec((S, DIM), lambda j: (0, 0)),
            pl.BlockSpec((S, DIM), lambda j: (0, 0)),
            pl.BlockSpec((DIM, CH), lambda j: (0, j)),
            pl.BlockSpec((CH, DIM), lambda j: (j, 0)),
        ],
        out_specs=pl.BlockSpec((S, DIM), lambda j: (0, 0)),
        out_shape=jax.ShapeDtypeStruct((S, DIM), f32),
    )(hbf, x2, W1.astype(bf16), W2.astype(bf16))

    return y.reshape(B, S, DIM)
